# baseline (speedup 1.0000x reference)
import functools

import jax
import jax.numpy as jnp
import numpy as np
from jax import lax
from jax.experimental import pallas as pl
from jax.experimental.pallas import tpu as pltpu
from jax.experimental.pallas import tpu_sc as plsc

B = 16384
D = 768
P = D // 2
G = 16
NC = 2
NS = 16
RW = B // NS
NPH = 3
PPP = 64
CT = 128
CTP = CT + 1
RC = 64
NJ = RW // (2 * RC)
TWP = PPP * G * G * 3

CLMAX = float(np.float32(np.float32(1.0) - np.float32(1e-6)) * np.float32(16.0))

_mesh = plsc.VectorSubcoreMesh(core_axis_name="c", subcore_axis_name="s")


@functools.partial(
    pl.kernel,
    mesh=_mesh,
    compiler_params=pltpu.CompilerParams(needs_layout_passes=False),
    out_type=jax.ShapeDtypeStruct((NC, B), jnp.float32),
    scratch_types=[
        pltpu.VMEM((TWP,), jnp.float32),
        pltpu.VMEM((RC, CTP), jnp.float32),
        pltpu.VMEM((RC, CTP), jnp.float32),
        pltpu.VMEM((1, RW), jnp.float32),
        pltpu.SemaphoreType.DMA,
        pltpu.SemaphoreType.DMA,
    ],
)
def _spline_partials(x_hbm, ctab_hbm, out_hbm, tab, xb0, xb1, ob,
                     semx0, semx1):
    cc = lax.axis_index("c")
    ss = lax.axis_index("s")
    row0 = ss * RW
    iota = lax.iota(jnp.int32, 16)
    zero16 = jnp.zeros((16,), jnp.float32)

    xbufs = (xb0, xb1)
    semx = (semx0, semx1)

    def zero_body(i, carry):
        ob[0, pl.ds(i * 16, 16)] = zero16
        return carry

    lax.fori_loop(0, RW // 16, zero_body, 0)

    def phase_body(t, carry):
        ct = cc * NPH + t

        pltpu.sync_copy(
            ctab_hbm.at[pl.ds(pl.multiple_of(ct * TWP, 128), TWP)], tab)

        def start_xchunk(m, buf_idx):
            pltpu.async_copy(
                x_hbm.at[pl.ds(pl.multiple_of(row0 + m * RC, RC), RC),
                         pl.ds(pl.multiple_of(ct * CT, CT), CT)],
                xbufs[buf_idx].at[pl.ds(0, RC), pl.ds(0, CT)],
                semx[buf_idx])

        def wait_x(buf_idx):
            pltpu.make_async_copy(
                x_hbm.at[pl.ds(0, RC), pl.ds(0, CT)],
                xbufs[buf_idx].at[pl.ds(0, RC), pl.ds(0, CT)],
                semx[buf_idx]).wait()

        start_xchunk(0, 0)
        start_xchunk(1, 1)

        def compute(buf, j, half):
            def r16_body(i, carry2):
                rows = i * 16 + iota
                obase = j * (2 * RC) + half * RC + i * 16
                acc = jnp.zeros((16,), jnp.float32)
                for u in range(PPP):
                    a = buf_gather(buf, rows, 2 * u)
                    b = buf_gather(buf, rows, 2 * u + 1)
                    fa = jnp.minimum(jnp.maximum(a * 8.0 + 8.0, 0.0), CLMAX)
                    fb = jnp.minimum(jnp.maximum(b * 8.0 + 8.0, 0.0), CLMAX)
                    ia = fa.astype(jnp.int32)
                    ib = fb.astype(jnp.int32)
                    idx = ia * 48 + ib * 3 + (u * G * G * 3)
                    c0 = idx.astype(jnp.float32)
                    c1 = (idx + 1).astype(jnp.float32)
                    c2 = (idx + 2).astype(jnp.float32)
                    an = fa * 0.0625
                    bn = fb * 0.0625
                    acc = acc + (c0 + c1 * an + c2 * bn)
                ob[0, pl.ds(obase, 16)] = ob[0, pl.ds(obase, 16)] + acc
                return carry2

            lax.fori_loop(0, RC // 16, r16_body, 0)

        def buf_gather(buf, rows, col):
            return plsc.load_gather(
                buf, [rows, jnp.full((16,), col, jnp.int32)])

        def body_j(j, carry2):
            wait_x(0)
            compute(xbufs[0], j, 0)

            @pl.when(j < NJ - 1)
            def _():
                start_xchunk(2 * j + 2, 0)

            wait_x(1)
            compute(xbufs[1], j, 1)

            @pl.when(j < NJ - 1)
            def _():
                start_xchunk(2 * j + 3, 1)

            return carry2

        lax.fori_loop(0, NJ, body_j, 0)
        return carry

    lax.fori_loop(0, NPH, phase_body, 0)

    pltpu.sync_copy(ob, out_hbm.at[pl.ds(cc, 1), pl.ds(row0, RW)])


def _reduce_body(p_ref, b_ref, o_ref):
    o_ref[...] = jnp.sum(p_ref[...], axis=0, keepdims=True) + b_ref[...]


def kernel(x, coeffs, bias):
    ctab = coeffs.reshape(P * G * G * 3)
    partials = _spline_partials(x, ctab)
    out = pl.pallas_call(
        _reduce_body,
        out_shape=jax.ShapeDtypeStruct((1, B), jnp.float32),
    )(partials, bias.reshape(1, 1))
    return out.reshape(B, 1)


# speedup vs baseline: 1.7053x; 1.7053x over previous
import functools

import jax
import jax.numpy as jnp
import numpy as np
from jax import lax
from jax.experimental import pallas as pl
from jax.experimental.pallas import tpu as pltpu
from jax.experimental.pallas import tpu_sc as plsc

B = 16384
D = 768
P = D // 2
G = 16
NW = 32
PPW = P // NW
CPW = 2 * PPW
TW = PPW * G * G * 3
R = 2048
NCHUNK = B // R
NR16 = R // 16

CLMAX = float(np.float32(np.float32(1.0) - np.float32(1e-6)) * np.float32(16.0))

_mesh = plsc.VectorSubcoreMesh(core_axis_name="c", subcore_axis_name="s")


@functools.partial(
    pl.kernel,
    mesh=_mesh,
    compiler_params=pltpu.CompilerParams(needs_layout_passes=False),
    out_type=jax.ShapeDtypeStruct((NW, B), jnp.float32),
    scratch_types=[
        pltpu.VMEM((TW,), jnp.float32),
        pltpu.VMEM((CPW, R), jnp.float32),
        pltpu.VMEM((CPW, R), jnp.float32),
        pltpu.VMEM((1, R), jnp.float32),
        pltpu.VMEM((1, R), jnp.float32),
        pltpu.SemaphoreType.DMA,
        pltpu.SemaphoreType.DMA,
        pltpu.SemaphoreType.DMA,
        pltpu.SemaphoreType.DMA,
    ],
)
def _spline_partials(xt_hbm, ctab_hbm, out_hbm, tab_v, xb0, xb1, ob0, ob1,
                     semh0, semh1, semo0, semo1):
    wid = lax.axis_index("s") * 2 + lax.axis_index("c")
    row0 = wid * CPW

    pltpu.sync_copy(
        ctab_hbm.at[pl.ds(pl.multiple_of(wid * TW, 128), TW)], tab_v)

    xbufs = (xb0, xb1)
    obufs = (ob0, ob1)
    semh = (semh0, semh1)
    semo = (semo0, semo1)
    copies = [None, None]
    ocp = [None, None]
    copies[0] = pltpu.async_copy(
        xt_hbm.at[pl.ds(row0, CPW), pl.ds(0, R)], xb0, semh0)

    for c in range(NCHUNK):
        s = c % 2
        if c + 1 < NCHUNK:
            copies[1 - s] = pltpu.async_copy(
                xt_hbm.at[pl.ds(row0, CPW), pl.ds((c + 1) * R, R)],
                xbufs[1 - s], semh[1 - s])
        copies[s].wait()
        buf = xbufs[s]
        ob = obufs[s]
        if ocp[s] is not None:
            ocp[s].wait()

        def r16_body(i, carry):
            acc = jnp.zeros((16,), jnp.float32)
            for dp in range(PPW):
                a = buf[2 * dp, pl.ds(i * 16, 16)]
                b = buf[2 * dp + 1, pl.ds(i * 16, 16)]
                fa = jnp.minimum(jnp.maximum(a * 8.0 + 8.0, 0.0), CLMAX)
                fb = jnp.minimum(jnp.maximum(b * 8.0 + 8.0, 0.0), CLMAX)
                ia = fa.astype(jnp.int32)
                ib = fb.astype(jnp.int32)
                idx = ia * 48 + ib * 3 + (dp * G * G * 3)
                c0 = plsc.load_gather(tab_v, [idx])
                c1 = plsc.load_gather(tab_v, [idx + 1])
                c2 = plsc.load_gather(tab_v, [idx + 2])
                an = fa * 0.0625
                bn = fb * 0.0625
                acc = acc + (c0 + c1 * an + c2 * bn)
            ob[0, pl.ds(i * 16, 16)] = acc
            return carry

        lax.fori_loop(0, NR16, r16_body, 0)
        ocp[s] = pltpu.async_copy(
            ob, out_hbm.at[pl.ds(wid, 1), pl.ds(c * R, R)], semo[s])

    for o in ocp:
        if o is not None:
            o.wait()


def _reduce_body(p_ref, b_ref, o_ref):
    o_ref[...] = jnp.sum(p_ref[...], axis=0, keepdims=True) + b_ref[...]


def kernel(x, coeffs, bias):
    ctab = coeffs.reshape(P * G * G * 3)
    partials = _spline_partials(x.T, ctab)
    out = pl.pallas_call(
        _reduce_body,
        out_shape=jax.ShapeDtypeStruct((1, B), jnp.float32),
    )(partials, bias.reshape(1, 1))
    return out.reshape(B, 1)
